# baseline (speedup 1.0000x reference)
import functools

import jax
import jax.numpy as jnp
from jax import lax
from jax.experimental import pallas as pl
from jax.experimental.pallas import tpu as pltpu
from jax.experimental.pallas import tpu_sc as plsc

N = 20000
MAX_OUT = 100
IOU_THR = 0.5
SCORE_THR = 0.05

NPAD = 20480
NL1 = NPAD // 16
NL2 = NL1 // 16
NL2V = NL2 // 16
KCAP = 112
KSL = KCAP // 16

_GDN = lax.GatherDimensionNumbers(
    offset_dims=(), collapsed_slice_dims=(0,), start_index_map=(0,))


def _perm(x, idx):
    return lax.gather(x, idx.reshape(16, 1), _GDN, (1,),
                      mode=lax.GatherScatterMode.PROMISE_IN_BOUNDS)


def _tree_max(x, lane):
    for k in (8, 4, 2, 1):
        x = jnp.maximum(x, _perm(x, lane ^ k))
    return x


def _nms_sc(bh, sh,
            obh, osch, oidxh,
            vb, vs, lvl1, lvl2,
            kx1, ky1, kx2, ky2, ka, kb, ksc, kidx,
            sem1):
    tile0 = jnp.logical_and(lax.axis_index("c") == 0, lax.axis_index("s") == 0)

    @pl.when(tile0)
    def _():
        lane = lax.iota(jnp.int32, 16)
        lane0 = lane == 0

        cb = pltpu.async_copy(bh, vb, sem1)
        pltpu.sync_copy(sh, vs.at[pl.ds(0, N)])

        neg16 = jnp.full((16,), -1.0, jnp.float32)
        for t in range((NPAD - N) // 16):
            vs[pl.ds(N + t * 16, 16)] = neg16

        zero16 = jnp.zeros((16,), jnp.float32)
        for t in range(KSL):
            kx1[pl.ds(t * 16, 16)] = zero16
            ky1[pl.ds(t * 16, 16)] = zero16
            kx2[pl.ds(t * 16, 16)] = zero16
            ky2[pl.ds(t * 16, 16)] = zero16
            ka[pl.ds(t * 16, 16)] = zero16
            ksc[pl.ds(t * 16, 16)] = zero16
            kidx[pl.ds(t * 16, 16)] = jnp.full((16,), -1, jnp.int32)
        for t in range(KSL * 4):
            kb[pl.ds(t * 16, 16)] = zero16

        def initb(i, _):
            base = i * 256
            mx = plsc.load_gather(vs, [base + lane * 16])
            for c in range(1, 16):
                mx = jnp.maximum(mx, plsc.load_gather(vs, [base + lane * 16 + c]))
            lvl1[pl.ds(i * 16, 16)] = mx
            return 0

        lax.fori_loop(0, NL2, initb, 0, unroll=4)

        l3 = jnp.full((16,), -1.0, jnp.float32)
        for t in range(NL2V):
            base = t * 256
            mx = plsc.load_gather(lvl1, [base + lane * 16])
            for c in range(1, 16):
                mx = jnp.maximum(mx, plsc.load_gather(lvl1, [base + lane * 16 + c]))
            lvl2[pl.ds(t * 16, 16)] = mx
            l3 = jnp.where(lane == t, _tree_max(mx, lane), l3)

        m0 = _tree_max(l3, lane)

        cb.wait()

        def cond(carry):
            k, ok = carry[0], carry[1]
            return jnp.logical_and(k < MAX_OUT, ok)

        def body(carry):
            k, _, m, l3 = carry
            gs = plsc.all_reduce_ffs(l3 == m)
            l2sel = plsc.load_gather(lvl2, [gs * 16 + lane])
            f = plsc.all_reduce_ffs(l2sel == m)
            g = gs * 16 + f
            lv = plsc.load_gather(lvl1, [g * 16 + lane])
            f2 = plsc.all_reduce_ffs(lv == m)
            j = g * 16 + f2
            sl = plsc.load_gather(vs, [j * 16 + lane])
            f3 = plsc.all_reduce_ffs(sl == m)
            idx = j * 16 + f3
            slp = jnp.where(lane == f3, jnp.float32(-1.0), sl)
            plsc.store_scatter(vs, [idx], jnp.full((16,), -1.0), mask=lane0)
            n1 = _tree_max(slp, lane)
            plsc.store_scatter(lvl1, [j], n1, mask=lane0)
            lvp = jnp.where(lane == f2, n1, lv)
            n2 = _tree_max(lvp, lane)
            plsc.store_scatter(lvl2, [g], n2, mask=lane0)
            l2p = jnp.where(lane == f, n2, l2sel)
            n3 = _tree_max(l2p, lane)
            l3 = jnp.where(lane == gs, n3, l3)
            m2 = _tree_max(l3, lane)
            ib = idx * 4
            bx1 = plsc.load_gather(vb, [ib])
            by1 = plsc.load_gather(vb, [ib + 1])
            bx2 = plsc.load_gather(vb, [ib + 2])
            by2 = plsc.load_gather(vb, [ib + 3])
            barea = (bx2 - bx1) * (by2 - by1)
            acc = jnp.zeros((16,), jnp.bool_)
            for t in range(KSL):
                tx1 = kx1[pl.ds(t * 16, 16)]
                ty1 = ky1[pl.ds(t * 16, 16)]
                tx2 = kx2[pl.ds(t * 16, 16)]
                ty2 = ky2[pl.ds(t * 16, 16)]
                ta = ka[pl.ds(t * 16, 16)]
                xx1 = jnp.maximum(tx1, bx1)
                yy1 = jnp.maximum(ty1, by1)
                xx2 = jnp.minimum(tx2, bx2)
                yy2 = jnp.minimum(ty2, by2)
                inter = (jnp.maximum(xx2 - xx1, 0.0)
                         * jnp.maximum(yy2 - yy1, 0.0))
                iou = inter / (ta + barea - inter + 1e-8)
                acc = jnp.logical_or(acc, iou >= IOU_THR)
            keep = jnp.logical_not(jnp.any(acc))

            @pl.when(keep)
            def _():
                kv = jnp.full((16,), k, jnp.int32)
                plsc.store_scatter(kx1, [kv], bx1, mask=lane0)
                plsc.store_scatter(ky1, [kv], by1, mask=lane0)
                plsc.store_scatter(kx2, [kv], bx2, mask=lane0)
                plsc.store_scatter(ky2, [kv], by2, mask=lane0)
                plsc.store_scatter(ka, [kv], barea, mask=lane0)
                kr = kv * 4
                plsc.store_scatter(kb, [kr], bx1, mask=lane0)
                plsc.store_scatter(kb, [kr + 1], by1, mask=lane0)
                plsc.store_scatter(kb, [kr + 2], bx2, mask=lane0)
                plsc.store_scatter(kb, [kr + 3], by2, mask=lane0)
                plsc.store_scatter(ksc, [kv], m, mask=lane0)
                plsc.store_scatter(kidx, [kv], idx, mask=lane0)

            k = k + keep.astype(jnp.int32)
            ok = jnp.any(m2 >= SCORE_THR)
            return (k, ok, m2, l3)

        lax.while_loop(cond, body,
                       (jnp.int32(0), jnp.any(m0 >= SCORE_THR), m0, l3))

        o1 = pltpu.async_copy(kb, obh, sem1)
        o1.wait()
        o2 = pltpu.async_copy(ksc, osch, sem1)
        o3 = pltpu.async_copy(kidx, oidxh, sem1)
        o2.wait()
        o3.wait()


def kernel(boxes, scores):
    mesh = plsc.VectorSubcoreMesh(
        core_axis_name="c", subcore_axis_name="s", num_cores=2, num_subcores=16)
    f32 = jnp.float32
    run = functools.partial(
        pl.kernel, mesh=mesh,
        compiler_params=pltpu.CompilerParams(needs_layout_passes=False),
        out_type=[jax.ShapeDtypeStruct((KCAP * 4,), f32),
                  jax.ShapeDtypeStruct((KCAP,), f32),
                  jax.ShapeDtypeStruct((KCAP,), jnp.int32)],
        scratch_types=[
            pltpu.VMEM((N * 4,), f32),
            pltpu.VMEM((NPAD,), f32),
            pltpu.VMEM((NL1,), f32),
            pltpu.VMEM((NL2,), f32),
            pltpu.VMEM((KCAP,), f32),
            pltpu.VMEM((KCAP,), f32),
            pltpu.VMEM((KCAP,), f32),
            pltpu.VMEM((KCAP,), f32),
            pltpu.VMEM((KCAP,), f32),
            pltpu.VMEM((KCAP * 4,), f32),
            pltpu.VMEM((KCAP,), f32),
            pltpu.VMEM((KCAP,), jnp.int32),
            pltpu.SemaphoreType.DMA,
        ],
    )(_nms_sc)
    ob, osc, oidx = run(boxes.reshape(N * 4), scores)
    kept_boxes = ob.reshape(KCAP, 4)[:MAX_OUT]
    return kept_boxes, osc[:MAX_OUT], oidx[:MAX_OUT]


# speedup vs baseline: 1.0136x; 1.0136x over previous
import functools

import jax
import jax.numpy as jnp
from jax import lax
from jax.experimental import pallas as pl
from jax.experimental.pallas import tpu as pltpu
from jax.experimental.pallas import tpu_sc as plsc

N = 20000
MAX_OUT = 100
IOU_THR = 0.5
SCORE_THR = 0.05

NPAD = 20480
NL1 = NPAD // 16
NL2 = NL1 // 16
NL2V = NL2 // 16
KCAP = 112
KSL = KCAP // 16

_GDN = lax.GatherDimensionNumbers(
    offset_dims=(), collapsed_slice_dims=(0,), start_index_map=(0,))


def _perm(x, idx):
    return lax.gather(x, idx.reshape(16, 1), _GDN, (1,),
                      mode=lax.GatherScatterMode.PROMISE_IN_BOUNDS)


def _tree_max(x, lane):
    for k in (8, 4, 2, 1):
        x = jnp.maximum(x, _perm(x, lane ^ k))
    return x


def _bmax(x, pin15):
    return _perm(plsc.cummax(x), pin15)


def _nms_sc(bh, sh,
            obh, osch, oidxh,
            vb, vs, lvl1, lvl2,
            kx1, ky1, kx2, ky2, ka, kb, ksc, kidx,
            sem1):
    tile0 = jnp.logical_and(lax.axis_index("c") == 0, lax.axis_index("s") == 0)

    @pl.when(tile0)
    def _():
        lane = lax.iota(jnp.int32, 16)
        lane0 = lane == 0
        pin15 = jnp.full((16,), 15, jnp.int32)

        cb = pltpu.async_copy(bh, vb, sem1)
        pltpu.sync_copy(sh, vs.at[pl.ds(0, N)])

        neg16 = jnp.full((16,), -1.0, jnp.float32)
        for t in range((NPAD - N) // 16):
            vs[pl.ds(N + t * 16, 16)] = neg16

        zero16 = jnp.zeros((16,), jnp.float32)
        for t in range(KSL):
            kx1[pl.ds(t * 16, 16)] = zero16
            ky1[pl.ds(t * 16, 16)] = zero16
            kx2[pl.ds(t * 16, 16)] = zero16
            ky2[pl.ds(t * 16, 16)] = zero16
            ka[pl.ds(t * 16, 16)] = zero16
            ksc[pl.ds(t * 16, 16)] = zero16
            kidx[pl.ds(t * 16, 16)] = jnp.full((16,), -1, jnp.int32)
        for t in range(KSL * 4):
            kb[pl.ds(t * 16, 16)] = zero16

        def initb(i, _):
            base = i * 256
            mx = plsc.load_gather(vs, [base + lane * 16])
            for c in range(1, 16):
                mx = jnp.maximum(mx, plsc.load_gather(vs, [base + lane * 16 + c]))
            lvl1[pl.ds(i * 16, 16)] = mx
            return 0

        lax.fori_loop(0, NL2, initb, 0, unroll=4)

        l3 = jnp.full((16,), -1.0, jnp.float32)
        for t in range(NL2V):
            base = t * 256
            mx = plsc.load_gather(lvl1, [base + lane * 16])
            for c in range(1, 16):
                mx = jnp.maximum(mx, plsc.load_gather(lvl1, [base + lane * 16 + c]))
            lvl2[pl.ds(t * 16, 16)] = mx
            l3 = jnp.where(lane == t, _bmax(mx, pin15), l3)

        m0 = _bmax(l3, pin15)

        cb.wait()

        def cond(carry):
            k, ok = carry[0], carry[1]
            return jnp.logical_and(k < MAX_OUT, ok)

        def body(carry):
            k, _, m, l3 = carry
            gs = plsc.all_reduce_ffs(l3 == m)
            l2sel = plsc.load_gather(lvl2, [gs * 16 + lane])
            f = plsc.all_reduce_ffs(l2sel == m)
            g = gs * 16 + f
            lv = plsc.load_gather(lvl1, [g * 16 + lane])
            f2 = plsc.all_reduce_ffs(lv == m)
            j = g * 16 + f2
            sl = plsc.load_gather(vs, [j * 16 + lane])
            f3 = plsc.all_reduce_ffs(sl == m)
            idx = j * 16 + f3
            slp = jnp.where(lane == f3, jnp.float32(-1.0), sl)
            plsc.store_scatter(vs, [idx], jnp.full((16,), -1.0), mask=lane0)
            n1 = _bmax(slp, pin15)
            plsc.store_scatter(lvl1, [j], n1, mask=lane0)
            lvp = jnp.where(lane == f2, n1, lv)
            n2 = _bmax(lvp, pin15)
            plsc.store_scatter(lvl2, [g], n2, mask=lane0)
            l2p = jnp.where(lane == f, n2, l2sel)
            n3 = _bmax(l2p, pin15)
            l3 = jnp.where(lane == gs, n3, l3)
            m2 = _bmax(l3, pin15)
            ib = idx * 4
            bx1 = plsc.load_gather(vb, [ib])
            by1 = plsc.load_gather(vb, [ib + 1])
            bx2 = plsc.load_gather(vb, [ib + 2])
            by2 = plsc.load_gather(vb, [ib + 3])
            barea = (bx2 - bx1) * (by2 - by1)
            acc = jnp.zeros((16,), jnp.bool_)
            for t in range(KSL):
                tx1 = kx1[pl.ds(t * 16, 16)]
                ty1 = ky1[pl.ds(t * 16, 16)]
                tx2 = kx2[pl.ds(t * 16, 16)]
                ty2 = ky2[pl.ds(t * 16, 16)]
                ta = ka[pl.ds(t * 16, 16)]
                xx1 = jnp.maximum(tx1, bx1)
                yy1 = jnp.maximum(ty1, by1)
                xx2 = jnp.minimum(tx2, bx2)
                yy2 = jnp.minimum(ty2, by2)
                inter = (jnp.maximum(xx2 - xx1, 0.0)
                         * jnp.maximum(yy2 - yy1, 0.0))
                iou = inter / (ta + barea - inter + 1e-8)
                acc = jnp.logical_or(acc, iou >= IOU_THR)
            keep = jnp.logical_not(jnp.any(acc))

            @pl.when(keep)
            def _():
                kv = jnp.full((16,), k, jnp.int32)
                plsc.store_scatter(kx1, [kv], bx1, mask=lane0)
                plsc.store_scatter(ky1, [kv], by1, mask=lane0)
                plsc.store_scatter(kx2, [kv], bx2, mask=lane0)
                plsc.store_scatter(ky2, [kv], by2, mask=lane0)
                plsc.store_scatter(ka, [kv], barea, mask=lane0)
                kr = kv * 4
                plsc.store_scatter(kb, [kr], bx1, mask=lane0)
                plsc.store_scatter(kb, [kr + 1], by1, mask=lane0)
                plsc.store_scatter(kb, [kr + 2], bx2, mask=lane0)
                plsc.store_scatter(kb, [kr + 3], by2, mask=lane0)
                plsc.store_scatter(ksc, [kv], m, mask=lane0)
                plsc.store_scatter(kidx, [kv], idx, mask=lane0)

            k = k + keep.astype(jnp.int32)
            ok = jnp.any(m2 >= SCORE_THR)
            return (k, ok, m2, l3)

        lax.while_loop(cond, body,
                       (jnp.int32(0), jnp.any(m0 >= SCORE_THR), m0, l3))

        o1 = pltpu.async_copy(kb, obh, sem1)
        o1.wait()
        o2 = pltpu.async_copy(ksc, osch, sem1)
        o3 = pltpu.async_copy(kidx, oidxh, sem1)
        o2.wait()
        o3.wait()


def kernel(boxes, scores):
    mesh = plsc.VectorSubcoreMesh(
        core_axis_name="c", subcore_axis_name="s", num_cores=2, num_subcores=16)
    f32 = jnp.float32
    run = functools.partial(
        pl.kernel, mesh=mesh,
        compiler_params=pltpu.CompilerParams(needs_layout_passes=False),
        out_type=[jax.ShapeDtypeStruct((KCAP * 4,), f32),
                  jax.ShapeDtypeStruct((KCAP,), f32),
                  jax.ShapeDtypeStruct((KCAP,), jnp.int32)],
        scratch_types=[
            pltpu.VMEM((N * 4,), f32),
            pltpu.VMEM((NPAD,), f32),
            pltpu.VMEM((NL1,), f32),
            pltpu.VMEM((NL2,), f32),
            pltpu.VMEM((KCAP,), f32),
            pltpu.VMEM((KCAP,), f32),
            pltpu.VMEM((KCAP,), f32),
            pltpu.VMEM((KCAP,), f32),
            pltpu.VMEM((KCAP,), f32),
            pltpu.VMEM((KCAP * 4,), f32),
            pltpu.VMEM((KCAP,), f32),
            pltpu.VMEM((KCAP,), jnp.int32),
            pltpu.SemaphoreType.DMA,
        ],
    )(_nms_sc)
    ob, osc, oidx = run(boxes.reshape(N * 4), scores)
    kept_boxes = ob.reshape(KCAP, 4)[:MAX_OUT]
    return kept_boxes, osc[:MAX_OUT], oidx[:MAX_OUT]


# speedup vs baseline: 1.0273x; 1.0135x over previous
import functools

import jax
import jax.numpy as jnp
from jax import lax
from jax.experimental import pallas as pl
from jax.experimental.pallas import tpu as pltpu
from jax.experimental.pallas import tpu_sc as plsc

N = 20000
MAX_OUT = 100
IOU_THR = 0.5
SCORE_THR = 0.05

NPAD = 20480
NL1 = NPAD // 16
NL2 = NL1 // 16
NL2V = NL2 // 16
KCAP = 112
KSL = KCAP // 16

_GDN = lax.GatherDimensionNumbers(
    offset_dims=(), collapsed_slice_dims=(0,), start_index_map=(0,))


def _perm(x, idx):
    return lax.gather(x, idx.reshape(16, 1), _GDN, (1,),
                      mode=lax.GatherScatterMode.PROMISE_IN_BOUNDS)


def _tree_max(x, lane):
    for k in (8, 4, 2, 1):
        x = jnp.maximum(x, _perm(x, lane ^ k))
    return x


def _bmax(x, pin15):
    return _perm(plsc.cummax(x), pin15)


def _nms_sc(bh, sh,
            obh, osch, oidxh,
            vb, vs, lvl1, lvl2,
            kx1, ky1, kx2, ky2, ka, kb, ksc, kidx,
            sem1):
    tile0 = jnp.logical_and(lax.axis_index("c") == 0, lax.axis_index("s") == 0)

    @pl.when(tile0)
    def _():
        lane = lax.iota(jnp.int32, 16)
        lane0 = lane == 0
        pin15 = jnp.full((16,), 15, jnp.int32)

        cb = pltpu.async_copy(bh, vb, sem1)
        pltpu.sync_copy(sh, vs.at[pl.ds(0, N)])

        neg16 = jnp.full((16,), -1.0, jnp.float32)
        for t in range((NPAD - N) // 16):
            vs[pl.ds(N + t * 16, 16)] = neg16

        zero16 = jnp.zeros((16,), jnp.float32)
        for t in range(KSL):
            kx1[pl.ds(t * 16, 16)] = zero16
            ky1[pl.ds(t * 16, 16)] = zero16
            kx2[pl.ds(t * 16, 16)] = zero16
            ky2[pl.ds(t * 16, 16)] = zero16
            ka[pl.ds(t * 16, 16)] = zero16
            ksc[pl.ds(t * 16, 16)] = zero16
            kidx[pl.ds(t * 16, 16)] = jnp.full((16,), -1, jnp.int32)
        for t in range(KSL * 4):
            kb[pl.ds(t * 16, 16)] = zero16

        def initb(i, _):
            base = i * 256
            mx = plsc.load_gather(vs, [base + lane * 16])
            for c in range(1, 16):
                mx = jnp.maximum(mx, plsc.load_gather(vs, [base + lane * 16 + c]))
            lvl1[pl.ds(i * 16, 16)] = mx
            return 0

        lax.fori_loop(0, NL2, initb, 0, unroll=4)

        l3 = jnp.full((16,), -1.0, jnp.float32)
        l2list = []
        for t in range(NL2V):
            base = t * 256
            mx = plsc.load_gather(lvl1, [base + lane * 16])
            for c in range(1, 16):
                mx = jnp.maximum(mx, plsc.load_gather(lvl1, [base + lane * 16 + c]))
            l2list.append(mx)
            l3 = jnp.where(lane == t, _bmax(mx, pin15), l3)

        m0 = _bmax(l3, pin15)

        cb.wait()

        def cond(carry):
            k, ok = carry[0], carry[1]
            return jnp.logical_and(k < MAX_OUT, ok)

        def body(carry):
            k, _, m, l3, l2r = carry
            gs = plsc.all_reduce_ffs(l3 == m)
            l2sel = l2r[NL2V - 1]
            for t in range(NL2V - 2, -1, -1):
                l2sel = jnp.where(gs == t, l2r[t], l2sel)
            f = plsc.all_reduce_ffs(l2sel == m)
            g = gs * 16 + f
            lv = plsc.load_gather(lvl1, [g * 16 + lane])
            f2 = plsc.all_reduce_ffs(lv == m)
            j = g * 16 + f2
            sl = plsc.load_gather(vs, [j * 16 + lane])
            f3 = plsc.all_reduce_ffs(sl == m)
            idx = j * 16 + f3
            slp = jnp.where(lane == f3, jnp.float32(-1.0), sl)
            plsc.store_scatter(vs, [idx], jnp.full((16,), -1.0), mask=lane0)
            n1 = _bmax(slp, pin15)
            plsc.store_scatter(lvl1, [j], n1, mask=lane0)
            lvp = jnp.where(lane == f2, n1, lv)
            n2 = _bmax(lvp, pin15)
            l2p = jnp.where(lane == f, n2, l2sel)
            l2r = tuple(jnp.where(gs == t, l2p, l2r[t]) for t in range(NL2V))
            n3 = _bmax(l2p, pin15)
            l3 = jnp.where(lane == gs, n3, l3)
            m2 = _bmax(l3, pin15)
            ib = idx * 4
            bx1 = plsc.load_gather(vb, [ib])
            by1 = plsc.load_gather(vb, [ib + 1])
            bx2 = plsc.load_gather(vb, [ib + 2])
            by2 = plsc.load_gather(vb, [ib + 3])
            barea = (bx2 - bx1) * (by2 - by1)
            acc = jnp.zeros((16,), jnp.bool_)
            for t in range(KSL):
                tx1 = kx1[pl.ds(t * 16, 16)]
                ty1 = ky1[pl.ds(t * 16, 16)]
                tx2 = kx2[pl.ds(t * 16, 16)]
                ty2 = ky2[pl.ds(t * 16, 16)]
                ta = ka[pl.ds(t * 16, 16)]
                xx1 = jnp.maximum(tx1, bx1)
                yy1 = jnp.maximum(ty1, by1)
                xx2 = jnp.minimum(tx2, bx2)
                yy2 = jnp.minimum(ty2, by2)
                inter = (jnp.maximum(xx2 - xx1, 0.0)
                         * jnp.maximum(yy2 - yy1, 0.0))
                iou = inter / (ta + barea - inter + 1e-8)
                acc = jnp.logical_or(acc, iou >= IOU_THR)
            keep = jnp.logical_not(jnp.any(acc))

            @pl.when(keep)
            def _():
                kv = jnp.full((16,), k, jnp.int32)
                plsc.store_scatter(kx1, [kv], bx1, mask=lane0)
                plsc.store_scatter(ky1, [kv], by1, mask=lane0)
                plsc.store_scatter(kx2, [kv], bx2, mask=lane0)
                plsc.store_scatter(ky2, [kv], by2, mask=lane0)
                plsc.store_scatter(ka, [kv], barea, mask=lane0)
                kr = kv * 4
                plsc.store_scatter(kb, [kr], bx1, mask=lane0)
                plsc.store_scatter(kb, [kr + 1], by1, mask=lane0)
                plsc.store_scatter(kb, [kr + 2], bx2, mask=lane0)
                plsc.store_scatter(kb, [kr + 3], by2, mask=lane0)
                plsc.store_scatter(ksc, [kv], m, mask=lane0)
                plsc.store_scatter(kidx, [kv], idx, mask=lane0)

            k = k + keep.astype(jnp.int32)
            ok = jnp.any(m2 >= SCORE_THR)
            return (k, ok, m2, l3, l2r)

        lax.while_loop(cond, body,
                       (jnp.int32(0), jnp.any(m0 >= SCORE_THR), m0, l3,
                        tuple(l2list)))

        o1 = pltpu.async_copy(kb, obh, sem1)
        o1.wait()
        o2 = pltpu.async_copy(ksc, osch, sem1)
        o3 = pltpu.async_copy(kidx, oidxh, sem1)
        o2.wait()
        o3.wait()


def kernel(boxes, scores):
    mesh = plsc.VectorSubcoreMesh(
        core_axis_name="c", subcore_axis_name="s", num_cores=2, num_subcores=16)
    f32 = jnp.float32
    run = functools.partial(
        pl.kernel, mesh=mesh,
        compiler_params=pltpu.CompilerParams(needs_layout_passes=False),
        out_type=[jax.ShapeDtypeStruct((KCAP * 4,), f32),
                  jax.ShapeDtypeStruct((KCAP,), f32),
                  jax.ShapeDtypeStruct((KCAP,), jnp.int32)],
        scratch_types=[
            pltpu.VMEM((N * 4,), f32),
            pltpu.VMEM((NPAD,), f32),
            pltpu.VMEM((NL1,), f32),
            pltpu.VMEM((NL2,), f32),
            pltpu.VMEM((KCAP,), f32),
            pltpu.VMEM((KCAP,), f32),
            pltpu.VMEM((KCAP,), f32),
            pltpu.VMEM((KCAP,), f32),
            pltpu.VMEM((KCAP,), f32),
            pltpu.VMEM((KCAP * 4,), f32),
            pltpu.VMEM((KCAP,), f32),
            pltpu.VMEM((KCAP,), jnp.int32),
            pltpu.SemaphoreType.DMA,
        ],
    )(_nms_sc)
    ob, osc, oidx = run(boxes.reshape(N * 4), scores)
    kept_boxes = ob.reshape(KCAP, 4)[:MAX_OUT]
    return kept_boxes, osc[:MAX_OUT], oidx[:MAX_OUT]
